# baseline (speedup 1.0000x reference)
import functools

import jax
import jax.numpy as jnp
from jax import lax
from jax.experimental import pallas as pl
from jax.experimental.pallas import tpu as pltpu
from jax.experimental.pallas import tpu_sc as plsc

_VOCAB = 1000
_D_EMBED = 128
_D_HIDDEN = 2048
_BATCH = 16384
_HALF = _BATCH // 2

_NC = 2
_NS = 16
_NW = _NC * _NS
_CHUNK = 128


_D_PACK = _D_EMBED // 2


@functools.cache
def _sc_gather(nrows, offset):
    b_per_w = nrows // _NW
    n_chunks = -(-b_per_w // _CHUNK)
    chunk = b_per_w // n_chunks
    assert n_chunks * chunk == b_per_w and chunk % 8 == 0

    def body(a_hbm, b_hbm, ea_hbm, eb_hbm, xa_hbm, xb_hbm,
             idx_a, idx_b, rows_a, rows_b, sem_ga, sem_gb, sem_sa, sem_sb):
        wid = lax.axis_index("s") * _NC + lax.axis_index("c")
        base = wid * b_per_w
        pltpu.sync_copy(a_hbm.at[pl.ds(offset + base, b_per_w)], idx_a)
        pltpu.sync_copy(b_hbm.at[pl.ds(offset + base, b_per_w)], idx_b)
        for j in range(n_chunks):
            off = base + j * chunk
            sl = pl.ds(j * chunk, chunk)
            ga = pltpu.async_copy(ea_hbm.at[idx_a.at[sl]], rows_a, sem_ga)
            gb = pltpu.async_copy(eb_hbm.at[idx_b.at[sl]], rows_b, sem_gb)
            ga.wait()
            sa = pltpu.async_copy(rows_a, xa_hbm.at[pl.ds(off, chunk)], sem_sa)
            gb.wait()
            sb = pltpu.async_copy(rows_b, xb_hbm.at[pl.ds(off, chunk)], sem_sb)
            sa.wait()
            sb.wait()

    return pl.kernel(
        body,
        out_type=(
            jax.ShapeDtypeStruct((nrows, _D_EMBED), jnp.float32),
            jax.ShapeDtypeStruct((nrows, _D_EMBED), jnp.float32),
        ),
        mesh=plsc.VectorSubcoreMesh(core_axis_name="c", subcore_axis_name="s"),
        scratch_types=(
            pltpu.VMEM((b_per_w,), jnp.int32),
            pltpu.VMEM((b_per_w,), jnp.int32),
            pltpu.VMEM((chunk, _D_EMBED), jnp.float32),
            pltpu.VMEM((chunk, _D_EMBED), jnp.float32),
            pltpu.SemaphoreType.DMA,
            pltpu.SemaphoreType.DMA,
            pltpu.SemaphoreType.DMA,
            pltpu.SemaphoreType.DMA,
        ),
    )


_BT = 1024
_DN_T = (((1,), (1,)), ((), ()))


def _mlp_body(xa_ref, xb_ref, wi_ref, wo_ref, out_ref):
    x = (xa_ref[...] + xb_ref[...]).astype(jnp.bfloat16)
    h = jnp.maximum(
        lax.dot_general(x, wi_ref[...], _DN_T,
                        preferred_element_type=jnp.float32), 0.0)
    out_ref[...] = lax.dot_general(wo_ref[...], h.astype(jnp.bfloat16), _DN_T,
                                   preferred_element_type=jnp.float32)


def _mlp_body_alias(xa_ref, xb_ref, wi_ref, wo_ref, prev_ref, out_ref):
    del prev_ref
    _mlp_body(xa_ref, xb_ref, wi_ref, wo_ref, out_ref)


def _mlp_call(xa, xb, wi, wo, out_prev, block_off):
    nb = xa.shape[0] // _BT
    in_specs = [
        pl.BlockSpec((_BT, _D_EMBED), lambda i: (i, 0)),
        pl.BlockSpec((_BT, _D_EMBED), lambda i: (i, 0)),
        pl.BlockSpec((_D_HIDDEN, _D_EMBED), lambda i: (0, 0)),
        pl.BlockSpec((_VOCAB, _D_HIDDEN), lambda i: (0, 0)),
    ]
    args = [xa, xb, wi, wo]
    io_alias = {}
    body = _mlp_body
    if out_prev is not None:
        in_specs.append(pl.BlockSpec(memory_space=pl.ANY))
        args.append(out_prev)
        io_alias = {4: 0}
        body = _mlp_body_alias
    return pl.pallas_call(
        body,
        grid=(nb,),
        in_specs=in_specs,
        out_specs=pl.BlockSpec((_VOCAB, _BT), lambda i: (0, i + block_off)),
        out_shape=jax.ShapeDtypeStruct((_VOCAB, _BATCH), jnp.float32),
        input_output_aliases=io_alias,
    )(*args)


_SPLIT = 2048


def kernel(a, b, E_a, E_b, W_in, W_out):
    a = a.astype(jnp.int32)
    b = b.astype(jnp.int32)
    wi = W_in.astype(jnp.bfloat16)
    wo = W_out.astype(jnp.bfloat16)
    rest = _BATCH - _SPLIT
    xa1, xb1 = _sc_gather(_SPLIT, 0)(a, b, E_a, E_b)
    xa2, xb2 = _sc_gather(rest, _SPLIT)(a, b, E_a, E_b)
    out1 = _mlp_call(xa1, xb1, wi, wo, None, 0)
    out2 = _mlp_call(xa2, xb2, wi, wo, out1, _SPLIT // _BT)
    return out2.T


# speedup vs baseline: 1.0328x; 1.0328x over previous
import functools

import jax
import jax.numpy as jnp
from jax import lax
from jax.experimental import pallas as pl
from jax.experimental.pallas import tpu as pltpu
from jax.experimental.pallas import tpu_sc as plsc

_VOCAB = 1000
_D_EMBED = 128
_D_HIDDEN = 2048
_BATCH = 16384
_HALF = _BATCH // 2

_NC = 2
_NS = 16
_NW = _NC * _NS
_CHUNK = 128


_D_PACK = _D_EMBED // 2


@functools.cache
def _sc_gather(nrows, offset):
    b_per_w = nrows // _NW
    n_chunks = -(-b_per_w // _CHUNK)
    chunk = b_per_w // n_chunks
    assert n_chunks * chunk == b_per_w and chunk % 8 == 0

    def body(a_hbm, b_hbm, ea_hbm, eb_hbm, xa_hbm, xb_hbm,
             idx_a, idx_b, rows_a, rows_b, sem_ga, sem_gb, sem_sa, sem_sb):
        wid = lax.axis_index("s") * _NC + lax.axis_index("c")
        base = wid * b_per_w
        pltpu.sync_copy(a_hbm.at[pl.ds(offset + base, b_per_w)], idx_a)
        pltpu.sync_copy(b_hbm.at[pl.ds(offset + base, b_per_w)], idx_b)
        for j in range(n_chunks):
            off = base + j * chunk
            sl = pl.ds(j * chunk, chunk)
            ga = pltpu.async_copy(ea_hbm.at[idx_a.at[sl]], rows_a, sem_ga)
            gb = pltpu.async_copy(eb_hbm.at[idx_b.at[sl]], rows_b, sem_gb)
            ga.wait()
            sa = pltpu.async_copy(rows_a, xa_hbm.at[pl.ds(off, chunk)], sem_sa)
            gb.wait()
            sb = pltpu.async_copy(rows_b, xb_hbm.at[pl.ds(off, chunk)], sem_sb)
            sa.wait()
            sb.wait()

    return pl.kernel(
        body,
        out_type=(
            jax.ShapeDtypeStruct((nrows, _D_EMBED), jnp.float32),
            jax.ShapeDtypeStruct((nrows, _D_EMBED), jnp.float32),
        ),
        mesh=plsc.VectorSubcoreMesh(core_axis_name="c", subcore_axis_name="s"),
        scratch_types=(
            pltpu.VMEM((b_per_w,), jnp.int32),
            pltpu.VMEM((b_per_w,), jnp.int32),
            pltpu.VMEM((chunk, _D_EMBED), jnp.float32),
            pltpu.VMEM((chunk, _D_EMBED), jnp.float32),
            pltpu.SemaphoreType.DMA,
            pltpu.SemaphoreType.DMA,
            pltpu.SemaphoreType.DMA,
            pltpu.SemaphoreType.DMA,
        ),
    )


_BT = 1024
_DN_T = (((1,), (1,)), ((), ()))


def _mlp_body(xa_ref, xb_ref, wi_ref, wo_ref, out_ref):
    x = (xa_ref[...] + xb_ref[...]).astype(jnp.bfloat16)
    h = jnp.maximum(
        lax.dot_general(x, wi_ref[...], _DN_T,
                        preferred_element_type=jnp.float32), 0.0)
    out_ref[...] = lax.dot_general(wo_ref[...], h.astype(jnp.bfloat16), _DN_T,
                                   preferred_element_type=jnp.float32)


def _mlp_body_alias(xa_ref, xb_ref, wi_ref, wo_ref, prev_ref, out_ref):
    del prev_ref
    _mlp_body(xa_ref, xb_ref, wi_ref, wo_ref, out_ref)


def _mlp_call(xa, xb, wi, wo, out_prev, block_off):
    nb = xa.shape[0] // _BT
    in_specs = [
        pl.BlockSpec((_BT, _D_EMBED), lambda i: (i, 0)),
        pl.BlockSpec((_BT, _D_EMBED), lambda i: (i, 0)),
        pl.BlockSpec((_D_HIDDEN, _D_EMBED), lambda i: (0, 0)),
        pl.BlockSpec((_VOCAB, _D_HIDDEN), lambda i: (0, 0)),
    ]
    args = [xa, xb, wi, wo]
    io_alias = {}
    body = _mlp_body
    if out_prev is not None:
        in_specs.append(pl.BlockSpec(memory_space=pl.ANY))
        args.append(out_prev)
        io_alias = {4: 0}
        body = _mlp_body_alias
    return pl.pallas_call(
        body,
        grid=(nb,),
        in_specs=in_specs,
        out_specs=pl.BlockSpec((_VOCAB, _BT), lambda i: (0, i + block_off)),
        out_shape=jax.ShapeDtypeStruct((_VOCAB, _BATCH), jnp.float32),
        input_output_aliases=io_alias,
    )(*args)


_SPLIT = 4096


def kernel(a, b, E_a, E_b, W_in, W_out):
    a = a.astype(jnp.int32)
    b = b.astype(jnp.int32)
    wi = W_in.astype(jnp.bfloat16)
    wo = W_out.astype(jnp.bfloat16)
    rest = _BATCH - _SPLIT
    xa1, xb1 = _sc_gather(_SPLIT, 0)(a, b, E_a, E_b)
    xa2, xb2 = _sc_gather(rest, _SPLIT)(a, b, E_a, E_b)
    out1 = _mlp_call(xa1, xb1, wi, wo, None, 0)
    out2 = _mlp_call(xa2, xb2, wi, wo, out1, _SPLIT // _BT)
    return out2.T
